# Initial kernel scaffold; baseline (speedup 1.0000x reference)
import jax
import jax.numpy as jnp
from jax.experimental import pallas as pl


def kernel(idx, W):
    raise NotImplementedError("write your pallas kernel here")


# speedup vs baseline: 1.5031x; 1.5031x over previous
import functools

import jax
import jax.numpy as jnp
from jax import lax
from jax.experimental import pallas as pl
from jax.experimental.pallas import tpu as pltpu
from jax.experimental.pallas import tpu_sc as plsc

_B, _T = 16384, 200
_N = _B * _T
_NW = 32
_PER_W = _N // _NW
_CHUNK = 4096
_NCHUNK = _PER_W // _CHUNK
_OUT_CHUNK = _CHUNK * 8
_VREGS = _OUT_CHUNK // 16


_GATHER_DN = lax.GatherDimensionNumbers(
    offset_dims=(), collapsed_slice_dims=(0,), start_index_map=(0,))


def _vgather(v, ind):
    return lax.gather(v, ind[:, None], _GATHER_DN, slice_sizes=(1,),
                      mode=lax.GatherScatterMode.PROMISE_IN_BOUNDS)


def _sc_body(idx_hbm, out_hbm, idx_v, out_v):
    wid = lax.axis_index("s") * 2 + lax.axis_index("c")
    base = wid * _PER_W

    lane = lax.iota(jnp.int32, 16)
    pair = lane >> 3
    shift = lane & 7

    def chunk_body(c, _):
        off = base + c * _CHUNK
        pltpu.sync_copy(idx_hbm.at[pl.ds(off, _CHUNK)], idx_v)

        def group_body(m, _):
            v = idx_v[pl.ds(16 * m, 16)]
            for j in range(8):
                u = _vgather(v, 2 * j + pair)
                bits = ((u >> shift) & 1).astype(jnp.float32)
                out_v[pl.ds(16 * (8 * m + j), 16)] = bits
            return 0

        lax.fori_loop(0, _CHUNK // 16, group_body, 0)
        pltpu.sync_copy(out_v, out_hbm.at[pl.ds(8 * off, _OUT_CHUNK)])
        return 0

    lax.fori_loop(0, _NCHUNK, chunk_body, 0)


@jax.jit
def _sc_lookup(idx_flat):
    mesh = plsc.VectorSubcoreMesh(core_axis_name="c", subcore_axis_name="s")
    f = functools.partial(
        pl.kernel,
        mesh=mesh,
        out_type=jax.ShapeDtypeStruct((_N * 8,), jnp.float32),
        scratch_types=[
            pltpu.VMEM((_CHUNK,), jnp.int32),
            pltpu.VMEM((_OUT_CHUNK,), jnp.float32),
        ],
    )(_sc_body)
    return f(idx_flat)


def kernel(idx, W):
    del W
    out_flat = _sc_lookup(idx.reshape(-1))
    return out_flat.reshape(idx.shape + (2, 4))


# speedup vs baseline: 1.5075x; 1.0029x over previous
import functools

import jax
import jax.numpy as jnp
from jax import lax
from jax.experimental import pallas as pl
from jax.experimental.pallas import tpu as pltpu
from jax.experimental.pallas import tpu_sc as plsc

_B, _T = 16384, 200
_N = _B * _T
_NW = 32
_PER_W = _N // _NW
_CHUNK = 4096
_NCHUNK = _PER_W // _CHUNK
_OUT_CHUNK = _CHUNK * 8
_VREGS = _OUT_CHUNK // 16


_GATHER_DN = lax.GatherDimensionNumbers(
    offset_dims=(), collapsed_slice_dims=(0,), start_index_map=(0,))


def _vgather(v, ind):
    return lax.gather(v, ind[:, None], _GATHER_DN, slice_sizes=(1,),
                      mode=lax.GatherScatterMode.PROMISE_IN_BOUNDS)


def _sc_body(idx_hbm, out_hbm, idx_v, out_v):
    wid = lax.axis_index("s") * 2 + lax.axis_index("c")
    base = wid * _PER_W

    lane = lax.iota(jnp.int32, 16)
    pair = lane >> 3
    shift = lane & 7

    @pl.loop(0, _NCHUNK)
    def chunk_body(c):
        off = base + c * _CHUNK
        pltpu.sync_copy(idx_hbm.at[pl.ds(off, _CHUNK)], idx_v)

        @plsc.parallel_loop(0, _CHUNK // 16, unroll=4)
        def group_body(m):
            v = idx_v[pl.ds(16 * m, 16)]
            for j in range(8):
                u = _vgather(v, 2 * j + pair)
                bits = ((u >> shift) & 1).astype(jnp.float32)
                out_v[pl.ds(16 * (8 * m + j), 16)] = bits

        pltpu.sync_copy(out_v, out_hbm.at[pl.ds(8 * off, _OUT_CHUNK)])


@jax.jit
def _sc_lookup(idx_flat):
    mesh = plsc.VectorSubcoreMesh(core_axis_name="c", subcore_axis_name="s")
    f = functools.partial(
        pl.kernel,
        mesh=mesh,
        out_type=jax.ShapeDtypeStruct((_N * 8,), jnp.float32),
        scratch_types=[
            pltpu.VMEM((_CHUNK,), jnp.int32),
            pltpu.VMEM((_OUT_CHUNK,), jnp.float32),
        ],
    )(_sc_body)
    return f(idx_flat)


def kernel(idx, W):
    del W
    out_flat = _sc_lookup(idx.reshape(-1))
    return out_flat.reshape(idx.shape + (2, 4))


# speedup vs baseline: 163.8907x; 108.7184x over previous
import functools

import jax
import jax.numpy as jnp
from jax import lax
from jax.experimental import pallas as pl
from jax.experimental.pallas import tpu as pltpu
from jax.experimental.pallas import tpu_sc as plsc

_NW = 32
_NH = 128
_NH_PER_W = _NH // _NW


def _sc_body(idx_hbm, out_hbm, idx_v, out_v):
    wid = lax.axis_index("s") * 2 + lax.axis_index("c")

    @pl.loop(0, _NH_PER_W)
    def nh_body(i):
        nh = wid * _NH_PER_W + i
        pltpu.sync_copy(idx_hbm.at[:, nh], idx_v)

        for c in range(2):

            @plsc.parallel_loop(0, 200, unroll=2)
            def t_body(t):
                th = t >> 3
                tl = t & 7
                for s in range(8):
                    v = idx_v[th, tl, pl.ds(16 * s, 16)]
                    for b in range(4):
                        bits = ((v >> (4 * c + b)) & 1).astype(jnp.float32)
                        out_v[t, b, pl.ds(16 * s, 16)] = bits

            pltpu.sync_copy(out_v, out_hbm.at[:, c, nh])


@jax.jit
def _sc_lookup(idx_q):
    mesh = plsc.VectorSubcoreMesh(core_axis_name="c", subcore_axis_name="s")
    f = functools.partial(
        pl.kernel,
        mesh=mesh,
        out_type=jax.ShapeDtypeStruct((200, 2, 128, 4, 128), jnp.float32),
        scratch_types=[
            pltpu.VMEM((25, 8, 128), jnp.int32),
            pltpu.VMEM((200, 4, 128), jnp.float32),
        ],
    )(_sc_body)
    return f(idx_q)


def kernel(idx, W):
    del W
    idx_q = idx.reshape(128, 128, 25, 8).transpose(2, 0, 3, 1)
    o5 = _sc_lookup(idx_q)
    return o5.transpose(2, 4, 0, 1, 3).reshape(16384, 200, 2, 4)


# speedup vs baseline: 224.2282x; 1.3682x over previous
import functools

import jax
import jax.numpy as jnp
from jax import lax
from jax.experimental import pallas as pl
from jax.experimental.pallas import tpu as pltpu
from jax.experimental.pallas import tpu_sc as plsc

_NW = 32
_NH = 128
_NH_PER_W = _NH // _NW


def _sc_body(idx_hbm, out_hbm, idx_v, out0, out1, sem0, sem1):
    wid = lax.axis_index("s") * 2 + lax.axis_index("c")
    bufs = (out0, out1)
    sems = (sem0, sem1)

    @pl.loop(0, _NH_PER_W)
    def nh_body(i):
        nh = wid * _NH_PER_W + i
        pltpu.sync_copy(idx_hbm.at[:, nh], idx_v)

        for c in range(2):
            for h in range(2):
                buf, sem = bufs[h], sems[h]
                store = pltpu.make_async_copy(
                    buf, out_hbm.at[pl.ds(100 * h, 100), c, nh], sem)

                if c == 1:
                    store.wait()
                else:

                    @pl.when(i > 0)
                    def _():
                        store.wait()

                @plsc.parallel_loop(0, 100, unroll=2)
                def t_body(t):
                    gt = t + 100 * h
                    th = gt >> 3
                    tl = gt & 7
                    for s in range(8):
                        v = idx_v[th, tl, pl.ds(16 * s, 16)]
                        for b in range(4):
                            bits = ((v >> (4 * c + b)) & 1).astype(jnp.float32)
                            buf[t, b, pl.ds(16 * s, 16)] = bits

                store.start()

    last = wid * _NH_PER_W + _NH_PER_W - 1
    for h in range(2):
        pltpu.make_async_copy(
            bufs[h], out_hbm.at[pl.ds(100 * h, 100), 1, last], sems[h]).wait()


@jax.jit
def _sc_lookup(idx_q):
    mesh = plsc.VectorSubcoreMesh(core_axis_name="c", subcore_axis_name="s")
    f = functools.partial(
        pl.kernel,
        mesh=mesh,
        out_type=jax.ShapeDtypeStruct((200, 2, 128, 4, 128), jnp.float32),
        scratch_types=[
            pltpu.VMEM((25, 8, 128), jnp.int32),
            pltpu.VMEM((100, 4, 128), jnp.float32),
            pltpu.VMEM((100, 4, 128), jnp.float32),
            pltpu.SemaphoreType.DMA,
            pltpu.SemaphoreType.DMA,
        ],
    )(_sc_body)
    return f(idx_q)


def kernel(idx, W):
    del W
    idx_q = idx.reshape(128, 128, 25, 8).transpose(2, 0, 3, 1)
    o5 = _sc_lookup(idx_q)
    return o5.transpose(2, 4, 0, 1, 3).reshape(16384, 200, 2, 4)
